# baseline (speedup 1.0000x reference)
import functools

import jax
import jax.numpy as jnp
from jax import lax
from jax.experimental import pallas as pl
from jax.experimental.pallas import tpu as pltpu
from jax.experimental.pallas import tpu_sc as plsc

_B, _S, _D = 4, 8192, 1024
_BSC = 3
_NC, _NS = 2, 16
_NW = _NC * _NS
_SPW = _S // _NW
_C = 8
_NCH = _SPW // _C
_NV = _C * _D // 16
_NR = 3

_mesh = plsc.VectorSubcoreMesh(core_axis_name="c", subcore_axis_name="s")

_scratch = (
    [pltpu.VMEM((_BSC, _C, _D), jnp.float32)] * _NR
    + [pltpu.VMEM((_C, _D), jnp.float32)] * 2
    + [pltpu.SemaphoreType.DMA] * (2 * _NR + 2)
)


@functools.partial(
    pl.kernel,
    out_type=jax.ShapeDtypeStruct((_BSC, _S, _D), jnp.float32),
    mesh=_mesh,
    scratch_types=_scratch,
)
def _pos_add_sc(x_hbm, tab_hbm, out_hbm, *scr):
    xb = scr[:_NR]
    pb = scr[_NR:_NR + 2]
    ld = scr[_NR + 2:2 * _NR + 2]
    st = scr[2 * _NR + 2:3 * _NR + 2]
    ps = scr[3 * _NR + 2:]

    wid = lax.axis_index("s") * _NC + lax.axis_index("c")
    s_base = wid * _SPW

    def s0(c):
        return s_base + c * _C

    def start_load(c, k):
        pltpu.async_copy(x_hbm.at[pl.ds(0, _BSC), pl.ds(s0(c), _C)], xb[k], ld[k])

    def wait_load(k):
        pltpu.make_async_copy(x_hbm.at[pl.ds(0, _BSC), pl.ds(0, _C)], xb[k], ld[k]).wait()

    def start_store(c, k):
        pltpu.async_copy(xb[k], out_hbm.at[:, pl.ds(s0(c), _C)], st[k])

    def wait_store(k):
        pltpu.make_async_copy(xb[k], out_hbm.at[:, pl.ds(0, _C)], st[k]).wait()

    def start_pos(c, q):
        pltpu.async_copy(tab_hbm.at[pl.ds(s0(c), _C)], pb[q], ps[q])

    def wait_pos(q):
        pltpu.make_async_copy(tab_hbm.at[pl.ds(0, _C)], pb[q], ps[q]).wait()

    def do_add(k, q):
        buf, pos = xb[k], pb[q]

        @plsc.parallel_loop(0, _NV, unroll=2)
        def add_vec(i):
            r = i >> 6
            j = (i & 63) * 16
            v = pos[r, pl.ds(j, 16)]
            for b in range(_BSC):
                plsc.addupdate(buf.at[b, r, pl.ds(j, 16)], v)

    def gen_iter(c, k, q, first):
        wait_pos(q)
        wait_load(k)
        do_add(k, q)
        start_store(c, k)

        @pl.when(c + 2 < _NCH)
        def _prefetch(c=c, k=k, q=q):
            if not first:
                wait_store((k + 2) % _NR)
            start_load(c + 2, (k + 2) % _NR)
            start_pos(c + 2, q)

    start_load(0, 0)
    start_load(1, 1)
    start_pos(0, 0)
    start_pos(1, 1)

    gen_iter(0, 0, 0, first=True)
    gen_iter(1, 1, 1, first=False)

    def group_body(g, carry):
        c_lo = 2 + 6 * g
        for i in range(6):
            gen_iter(c_lo + i, (2 + i) % _NR, i % 2, first=False)
        return carry

    lax.fori_loop(0, (_NCH - 2) // 6, group_body, 0)

    for k in range(_NR):
        wait_store(k)


_TCS = 512


def _tc_body(x_ref, tab_ref, out_ref):
    out_ref[0] = x_ref[0] + tab_ref[...]


_tc_add = pl.pallas_call(
    _tc_body,
    grid=(_S // _TCS,),
    in_specs=[
        pl.BlockSpec((1, _TCS, _D), lambda s: (_BSC, s, 0)),
        pl.BlockSpec((_TCS, _D), lambda s: (s, 0)),
    ],
    out_specs=pl.BlockSpec((1, _TCS, _D), lambda s: (0, s, 0)),
    out_shape=jax.ShapeDtypeStruct((1, _S, _D), jnp.float32),
)


def kernel(x, pos_table):
    sc_part = _pos_add_sc(x, pos_table)
    tc_part = _tc_add(x, pos_table)
    return jnp.concatenate([sc_part, tc_part], axis=0)


# speedup vs baseline: 1.7059x; 1.7059x over previous
import functools

import jax
import jax.numpy as jnp
from jax import lax
from jax.experimental import pallas as pl
from jax.experimental.pallas import tpu as pltpu
from jax.experimental.pallas import tpu_sc as plsc

_B, _S, _D = 4, 8192, 1024
_NC, _NS = 2, 16
_NW = _NC * _NS
_SPW = _S // _NW
_C = 8
_NCH = _SPW // _C
_NV = _C * _D // 16
_NB = 8

_mesh = plsc.VectorSubcoreMesh(core_axis_name="c", subcore_axis_name="s")

_scratch = (
    [pltpu.VMEM((_C, _D), jnp.float32)] * (_NB + 2)
    + [pltpu.SemaphoreType.DMA] * (2 * _NB + 2)
)


@functools.partial(
    pl.kernel,
    out_type=jax.ShapeDtypeStruct((_B, _S, _D), jnp.float32),
    mesh=_mesh,
    scratch_types=_scratch,
)
def _pos_add(x_hbm, tab_hbm, out_hbm, *scr):
    xb = scr[:_NB]
    pb = scr[_NB:_NB + 2]
    ld = scr[_NB + 2:2 * _NB + 2]
    st = scr[2 * _NB + 2:3 * _NB + 2]
    ps = scr[3 * _NB + 2:]

    wid = lax.axis_index("s") * _NC + lax.axis_index("c")
    s_base = wid * _SPW

    def s0(c):
        return s_base + c * _C

    def start_load(c, b, k):
        pltpu.async_copy(x_hbm.at[b, pl.ds(s0(c), _C)], xb[k], ld[k])

    def wait_load(k):
        pltpu.make_async_copy(x_hbm.at[0, pl.ds(0, _C)], xb[k], ld[k]).wait()

    def start_store(c, b, k):
        pltpu.async_copy(xb[k], out_hbm.at[b, pl.ds(s0(c), _C)], st[k])

    def wait_store(k):
        pltpu.make_async_copy(xb[k], out_hbm.at[0, pl.ds(0, _C)], st[k]).wait()

    def start_pos(c, q):
        pltpu.async_copy(tab_hbm.at[pl.ds(s0(c), _C)], pb[q], ps[q])

    def wait_pos(q):
        pltpu.make_async_copy(tab_hbm.at[pl.ds(0, _C)], pb[q], ps[q]).wait()

    def do_add(k, q):
        buf, pos = xb[k], pb[q]

        @plsc.parallel_loop(0, _NV, unroll=4)
        def add_vec(i):
            r = i >> 6
            j = (i & 63) * 16
            plsc.addupdate(buf.at[r, pl.ds(j, 16)], pos[r, pl.ds(j, 16)])

    def gen_iter(c, cc, b, peeled_first):
        k = 4 * cc + b
        wait_load(k)
        do_add(k, cc)
        start_store(c, b, k)
        k2 = (k + 6) % 8
        cp, bp = (c + 1, b + 2) if b < 2 else (c + 2, b - 2)

        def issue():
            if not peeled_first:
                wait_store(k2)
            start_load(cp, bp, k2)

        if peeled_first or isinstance(cp, int):
            issue()
        else:
            pl.when(cp < _NCH)(issue)

    for b in range(_B):
        start_load(0, b, b)
    start_load(1, 0, 4)
    start_load(1, 1, 5)
    start_pos(0, 0)
    start_pos(1, 1)

    for c in (0, 1):
        wait_pos(c)
        for b in range(_B):
            gen_iter(c, c, b, peeled_first=(c == 0 and b < 2))
        start_pos(c + 2, c)

    def chunk_pair(c2, carry):
        for cc in (0, 1):
            c = 2 * c2 + cc
            wait_pos(cc)
            for b in range(_B):
                gen_iter(c, cc, b, peeled_first=False)

            @pl.when(c + 2 < _NCH)
            def _pos_prefetch(c=c, cc=cc):
                start_pos(c + 2, cc)
        return carry

    lax.fori_loop(1, _NCH // 2, chunk_pair, 0)

    for k in range(_NB):
        wait_store(k)


def kernel(x, pos_table):
    return _pos_add(x, pos_table)
